# Initial kernel scaffold; baseline (speedup 1.0000x reference)
import jax
import jax.numpy as jnp
from jax.experimental import pallas as pl


def kernel(x1, x2, edge_index1, edge_index2, edge_attr1, edge_attr2, embed, edge_embed, W_msg, b_msg, w_ih, w_hh, b_ih, b_hh, W_gate, b_gate):
    raise NotImplementedError("write your pallas kernel here")


# speedup vs baseline: 2.9895x; 2.9895x over previous
import functools

import jax
import jax.numpy as jnp
from jax import lax
from jax.experimental import pallas as pl
from jax.experimental.pallas import tpu as pltpu
from jax.experimental.pallas import tpu_sc as plsc

N_REAL = 10000
NP = 10240
E = 160000
D = 128
NC, NS = 2, 16
EPT = E // NS
K = 80
NCHUNK = EPT // K

@functools.cache
def _mesh():
    return plsc.VectorSubcoreMesh(
        core_axis_name="c", subcore_axis_name="s",
        num_cores=NC, num_subcores=NS)


@functools.cache
def _make_embed_gather():
    B = 2 * NP
    NW = NC * NS
    bpw = B // NW
    CH = 128

    @functools.partial(
        pl.kernel,
        out_type=jax.ShapeDtypeStruct((B, D), jnp.float32),
        mesh=_mesh(),
        scratch_types=[
            pltpu.VMEM((CH,), jnp.int32),
            pltpu.VMEM((CH, D), jnp.float32),
            pltpu.SemaphoreType.DMA,
        ],
    )
    def k(table_hbm, idx_hbm, out_hbm, idx_v, rows_v, sem):
        wid = lax.axis_index("s") * NC + lax.axis_index("c")
        base = wid * bpw

        def chunk(i, carry):
            off = base + i * CH
            pltpu.sync_copy(idx_hbm.at[pl.ds(off, CH)], idx_v)
            pltpu.async_copy(table_hbm.at[idx_v], rows_v, sem).wait()
            pltpu.sync_copy(rows_v, out_hbm.at[pl.ds(off, CH)])
            return carry

        lax.fori_loop(0, bpw // CH, chunk, 0)

    return k


def _embed_gather(*a):
    return _make_embed_gather()(*a)


def _proj_body(h_ref, w1_ref, w2_ref, ee_ref, w3_ref, bm_ref,
               a_ref, b_ref, c16_ref):
    i = pl.program_id(0)
    h = h_ref[...]
    a_ref[...] = jnp.dot(h, w1_ref[...], preferred_element_type=jnp.float32)
    b_ref[...] = jnp.dot(h, w2_ref[...], preferred_element_type=jnp.float32)

    @pl.when(i == 0)
    def _():
        c16_ref[...] = (
            jnp.dot(ee_ref[...], w3_ref[...], preferred_element_type=jnp.float32)
            + bm_ref[...])


def _proj(hp, w1t, w2t, edge_embed, w3t, b_msg):
    Bg = 512
    n = (2 * NP) // Bg
    full = lambda i: (0, 0)
    return pl.pallas_call(
        _proj_body,
        grid=(n,),
        in_specs=[
            pl.BlockSpec((Bg, D), lambda i: (i, 0)),
            pl.BlockSpec((D, D), full),
            pl.BlockSpec((D, D), full),
            pl.BlockSpec((16, D), full),
            pl.BlockSpec((D, D), full),
            pl.BlockSpec((1, D), full),
        ],
        out_specs=[
            pl.BlockSpec((Bg, D), lambda i: (i, 0)),
            pl.BlockSpec((Bg, D), lambda i: (i, 0)),
            pl.BlockSpec((16, D), full),
        ],
        out_shape=[
            jax.ShapeDtypeStruct((2 * NP, D), jnp.float32),
            jax.ShapeDtypeStruct((2 * NP, D), jnp.float32),
            jax.ShapeDtypeStruct((16, D), jnp.float32),
        ],
    )(hp, w1t, w2t, edge_embed, w3t, b_msg)


@functools.cache
def _make_msg_kernel():
    @functools.partial(
        pl.kernel,
        out_type=jax.ShapeDtypeStruct((2 * NP, D), jnp.float32),
        mesh=_mesh(),
        scratch_types=[
            pltpu.VMEM_SHARED((NP, D), jnp.float32),
            pltpu.VMEM((K,), jnp.int32),
            pltpu.VMEM((K,), jnp.int32),
            pltpu.VMEM((K,), jnp.int32),
            pltpu.VMEM((K,), jnp.int32),
            pltpu.VMEM((K, D), jnp.float32),
            pltpu.VMEM((K, D), jnp.float32),
            pltpu.VMEM((K, D), jnp.float32),
            pltpu.SemaphoreType.DMA,
            pltpu.SemaphoreType.DMA,
            pltpu.SemaphoreType.DMA,
        ],
    )
    def k(a_hbm, b_hbm, c16_hbm, dsto_hbm, srco_hbm, dstr_hbm, attr_hbm,
          out_hbm, macc, dstov, srcov, dstrv, attrv, av, bv, cv,
          sem_a, sem_b, sem_c):
        c = lax.axis_index("c")
        s = lax.axis_index("s")
        zero16 = jnp.zeros((16,), jnp.float32)

        def zrow(r, carry):
            for t in range(D // 16):
                av[r, pl.ds(t * 16, 16)] = zero16
            return carry

        lax.fori_loop(0, K, zrow, 0)

        def zslice(t, carry):
            pltpu.sync_copy(av, macc.at[pl.ds(s * 640 + t * K, K)])
            return carry

        lax.fori_loop(0, 640 // K, zslice, 0)
        plsc.subcore_barrier()

        def chunk(j, carry):
            pltpu.sync_copy(dsto_hbm.at[c, s, j], dstov)
            pltpu.sync_copy(srco_hbm.at[c, s, j], srcov)
            pltpu.sync_copy(dstr_hbm.at[c, s, j], dstrv)
            pltpu.sync_copy(attr_hbm.at[c, s, j], attrv)
            cp_a = pltpu.async_copy(a_hbm.at[dstov], av, sem_a)
            cp_b = pltpu.async_copy(b_hbm.at[srcov], bv, sem_b)
            cp_c = pltpu.async_copy(c16_hbm.at[attrv], cv, sem_c)
            cp_a.wait()
            cp_b.wait()
            cp_c.wait()

            def row(r, rcarry):
                for t in range(D // 16):
                    sl = pl.ds(t * 16, 16)
                    av[r, sl] = jnp.maximum(
                        av[r, sl] + bv[r, sl] + cv[r, sl], 0.0)
                return rcarry

            lax.fori_loop(0, K, row, 0)
            pltpu.sync_copy(av, macc.at[dstrv], add=True)
            return carry

        lax.fori_loop(0, NCHUNK, chunk, 0)
        plsc.subcore_barrier()

        def wb(t, carry):
            off = s * 640 + t * K
            pltpu.sync_copy(macc.at[pl.ds(off, K)], bv)
            pltpu.sync_copy(bv, out_hbm.at[pl.ds(c * NP + off, K)])
            return carry

        lax.fori_loop(0, 640 // K, wb, 0)

    return k


def _msg_kernel(*a):
    return _make_msg_kernel()(*a)


def _make_flash(Bq, Bk):
    nk = NP // Bk

    def body(q_ref, kv_ref, o_ref, acc_ref, l_ref):
        j = pl.program_id(1)

        @pl.when(j == 0)
        def _():
            acc_ref[...] = jnp.zeros_like(acc_ref)
            l_ref[...] = jnp.zeros_like(l_ref)

        q = q_ref[...]
        kv = kv_ref[...]
        s = lax.dot_general(q, kv, (((1,), (1,)), ((), ())),
                            preferred_element_type=jnp.float32)
        col = j * Bk + lax.broadcasted_iota(jnp.int32, (Bq, Bk), 1)
        e = jnp.where(col < N_REAL, jnp.exp(s), 0.0)
        acc_ref[...] += jnp.dot(e, kv, preferred_element_type=jnp.float32)
        l_ref[...] += jnp.sum(e, axis=1, keepdims=True)

        @pl.when(j == nk - 1)
        def _():
            o_ref[...] = q - acc_ref[...] / l_ref[...]

    return pl.pallas_call(
        body,
        grid=(NP // Bq, nk),
        in_specs=[
            pl.BlockSpec((Bq, D), lambda i, j: (i, 0)),
            pl.BlockSpec((Bk, D), lambda i, j: (j, 0)),
        ],
        out_specs=pl.BlockSpec((Bq, D), lambda i, j: (i, 0)),
        out_shape=jax.ShapeDtypeStruct((NP, D), jnp.float32),
        scratch_shapes=[
            pltpu.VMEM((Bq, D), jnp.float32),
            pltpu.VMEM((Bq, 128), jnp.float32),
        ],
    )


_flash = _make_flash(512, 512)


def _make_gru_pool(Bg):
    ng = NP // Bg

    def body(m_ref, u_ref, h_ref, wim_ref, wiu_ref, whh_ref, bih_ref,
             bhh_ref, wg_ref, bg_ref, o_ref, num_ref, den_ref):
        i = pl.program_id(0)

        @pl.when(i == 0)
        def _():
            num_ref[...] = jnp.zeros_like(num_ref)
            den_ref[...] = jnp.zeros_like(den_ref)

        m = m_ref[...]
        u = u_ref[...]
        h = h_ref[...]
        gi = (jnp.dot(m, wim_ref[...], preferred_element_type=jnp.float32)
              + jnp.dot(u, wiu_ref[...], preferred_element_type=jnp.float32)
              + bih_ref[...])
        gh = (jnp.dot(h, whh_ref[...], preferred_element_type=jnp.float32)
              + bhh_ref[...])
        r = jax.nn.sigmoid(gi[:, :D] + gh[:, :D])
        z = jax.nn.sigmoid(gi[:, D:2 * D] + gh[:, D:2 * D])
        n = jnp.tanh(gi[:, 2 * D:] + r * gh[:, 2 * D:])
        hn = (1.0 - z) * n + z * h

        gate = jax.nn.sigmoid(
            jnp.dot(hn, wg_ref[...], preferred_element_type=jnp.float32)
            + bg_ref[...])
        rowid = i * Bg + lax.broadcasted_iota(jnp.int32, (Bg, 1), 0)
        eg = jnp.where(rowid < N_REAL, jnp.exp(gate), 0.0)
        num_ref[...] += jnp.sum(eg * hn, axis=0, keepdims=True)
        den_ref[...] += jnp.sum(eg)

        @pl.when(i == ng - 1)
        def _():
            o_ref[...] = num_ref[...] / den_ref[...]

    full = lambda i: (0, 0)
    return pl.pallas_call(
        body,
        grid=(ng,),
        in_specs=[
            pl.BlockSpec((Bg, D), lambda i: (i, 0)),
            pl.BlockSpec((Bg, D), lambda i: (i, 0)),
            pl.BlockSpec((Bg, D), lambda i: (i, 0)),
            pl.BlockSpec((D, 3 * D), full),
            pl.BlockSpec((D, 3 * D), full),
            pl.BlockSpec((D, 3 * D), full),
            pl.BlockSpec((1, 3 * D), full),
            pl.BlockSpec((1, 3 * D), full),
            pl.BlockSpec((D, 1), full),
            pl.BlockSpec((1, 1), full),
        ],
        out_specs=pl.BlockSpec((1, D), full),
        out_shape=jax.ShapeDtypeStruct((1, D), jnp.float32),
        scratch_shapes=[
            pltpu.VMEM((1, D), jnp.float32),
            pltpu.VMEM((1, 128), jnp.float32),
        ],
    )


_gru_pool = _make_gru_pool(512)


def kernel(x1, x2, edge_index1, edge_index2, edge_attr1, edge_attr2,
           embed, edge_embed, W_msg, b_msg, w_ih, w_hh, b_ih, b_hh,
           W_gate, b_gate):
    i32 = jnp.int32
    pad = jnp.zeros((NP - N_REAL,), i32)
    idx = jnp.concatenate([
        x1[:, 0].astype(i32), pad, x2[:, 0].astype(i32), pad])

    hp = _embed_gather(embed, idx)

    w1t = W_msg[:, :D].T
    w2t = W_msg[:, D:2 * D].T
    w3t = W_msg[:, 2 * D:].T
    a_all, b_all, c16 = _proj(hp, w1t, w2t, edge_embed,
                              w3t, b_msg.reshape(1, D))

    def edge_layout(ei, ea):
        dst = ei[1].astype(i32)
        src = ei[0].astype(i32)
        at = ea[:, 0].astype(i32)
        shp = (NS, NCHUNK, K)
        return (dst.reshape(shp), src.reshape(shp), at.reshape(shp))

    d1, s1, a1 = edge_layout(edge_index1, edge_attr1)
    d2, s2, a2 = edge_layout(edge_index2, edge_attr2)
    dst_raw = jnp.stack([d1, d2])
    off = jnp.stack([jnp.zeros_like(d1), jnp.full_like(d2, NP)])
    dst_off = dst_raw + off
    src_off = jnp.stack([s1, s2]) + off
    attr = jnp.stack([a1, a2])

    m_all = _msg_kernel(a_all, b_all, c16, dst_off, src_off, dst_raw, attr)

    h1p = hp[:NP]
    h2p = hp[NP:]
    u1 = _flash(h1p, h2p)
    u2 = _flash(h2p, h1p)

    wim = w_ih[:, :D].T
    wiu = w_ih[:, D:].T
    whh = w_hh.T
    bih = b_ih.reshape(1, 3 * D)
    bhh = b_hh.reshape(1, 3 * D)
    wg = W_gate.T
    bg = b_gate.reshape(1, 1)

    hg1 = _gru_pool(m_all[:NP], u1, h1p, wim, wiu, whh, bih, bhh, wg, bg)
    hg2 = _gru_pool(m_all[NP:], u2, h2p, wim, wiu, whh, bih, bhh, wg, bg)
    return (hg1, hg2)


# speedup vs baseline: 4.4298x; 1.4818x over previous
import functools

import jax
import jax.numpy as jnp
from jax import lax
from jax.experimental import pallas as pl
from jax.experimental.pallas import tpu as pltpu
from jax.experimental.pallas import tpu_sc as plsc

N_REAL = 10000
NP = 10240
E = 160000
D = 128
NC, NS = 2, 16
EPT = E // NS
K = 80
NCHUNK = EPT // K
BLK = 25
NBLOCKS = NCHUNK // BLK
PAIRS = BLK // 2

@functools.cache
def _mesh():
    return plsc.VectorSubcoreMesh(
        core_axis_name="c", subcore_axis_name="s",
        num_cores=NC, num_subcores=NS)


@functools.cache
def _make_embed_gather():
    B = 2 * NP
    NW = NC * NS
    bpw = B // NW
    CH = 128

    @functools.partial(
        pl.kernel,
        out_type=jax.ShapeDtypeStruct((B, D), jnp.float32),
        mesh=_mesh(),
        scratch_types=[
            pltpu.VMEM((CH,), jnp.int32),
            pltpu.VMEM((CH, D), jnp.float32),
            pltpu.SemaphoreType.DMA,
        ],
    )
    def k(table_hbm, idx_hbm, out_hbm, idx_v, rows_v, sem):
        wid = lax.axis_index("s") * NC + lax.axis_index("c")
        base = wid * bpw

        def chunk(i, carry):
            off = base + i * CH
            pltpu.sync_copy(idx_hbm.at[pl.ds(off, CH)], idx_v)
            pltpu.async_copy(table_hbm.at[idx_v], rows_v, sem).wait()
            pltpu.sync_copy(rows_v, out_hbm.at[pl.ds(off, CH)])
            return carry

        lax.fori_loop(0, bpw // CH, chunk, 0)

    return k


def _embed_gather(*a):
    return _make_embed_gather()(*a)


def _proj_body(h_ref, w1_ref, w2_ref, ee_ref, w3_ref, bm_ref,
               a_ref, b_ref, c16_ref):
    i = pl.program_id(0)
    h = h_ref[...]
    a_ref[...] = jnp.dot(h, w1_ref[...], preferred_element_type=jnp.float32)
    b_ref[...] = jnp.dot(h, w2_ref[...], preferred_element_type=jnp.float32)

    @pl.when(i == 0)
    def _():
        c16_ref[...] = (
            jnp.dot(ee_ref[...], w3_ref[...], preferred_element_type=jnp.float32)
            + bm_ref[...])


def _proj(hp, w1t, w2t, edge_embed, w3t, b_msg):
    Bg = 512
    n = (2 * NP) // Bg
    full = lambda i: (0, 0)
    return pl.pallas_call(
        _proj_body,
        grid=(n,),
        in_specs=[
            pl.BlockSpec((Bg, D), lambda i: (i, 0)),
            pl.BlockSpec((D, D), full),
            pl.BlockSpec((D, D), full),
            pl.BlockSpec((16, D), full),
            pl.BlockSpec((D, D), full),
            pl.BlockSpec((1, D), full),
        ],
        out_specs=[
            pl.BlockSpec((Bg, D), lambda i: (i, 0)),
            pl.BlockSpec((Bg, D), lambda i: (i, 0)),
            pl.BlockSpec((16, D), full),
        ],
        out_shape=[
            jax.ShapeDtypeStruct((2 * NP, D), jnp.float32),
            jax.ShapeDtypeStruct((2 * NP, D), jnp.float32),
            jax.ShapeDtypeStruct((16, D), jnp.float32),
        ],
    )(hp, w1t, w2t, edge_embed, w3t, b_msg)


@functools.cache
def _make_msg_kernel():
    @functools.partial(
        pl.kernel,
        out_type=jax.ShapeDtypeStruct((2 * NP, D), jnp.float32),
        mesh=_mesh(),
        scratch_types=[
            pltpu.VMEM_SHARED((NP, D), jnp.float32),
            pltpu.VMEM((BLK, K), jnp.int32),
            pltpu.VMEM((BLK, K), jnp.int32),
            pltpu.VMEM((2, K), jnp.int32),
            pltpu.VMEM((K,), jnp.int32),
            pltpu.VMEM((16, D), jnp.float32),
            pltpu.VMEM((K, D), jnp.float32),
            pltpu.VMEM((K, D), jnp.float32),
            pltpu.VMEM((K, D), jnp.float32),
            pltpu.SemaphoreType.DMA,
            pltpu.SemaphoreType.DMA,
            pltpu.SemaphoreType.DMA,
            pltpu.SemaphoreType.DMA,
            pltpu.SemaphoreType.DMA,
        ],
    )
    def k(a_hbm, b_hbm, c16_hbm, dstr_hbm, srcr_hbm,
          out_hbm, macc, dstrv, srcrv, ixa, ixb, c16v,
          av0, av1, bv, sa0, sa1, sb, ss0, ss1):
        c = lax.axis_index("c")
        s = lax.axis_index("s")
        goff = c * NP
        zero16 = jnp.zeros((16,), jnp.float32)

        def zrow(r, carry):
            for t in range(D // 16):
                av0[r, pl.ds(t * 16, 16)] = zero16
            return carry

        lax.fori_loop(0, 40, zrow, 0)

        def zslice(t, carry):
            pltpu.sync_copy(av0.at[pl.ds(0, 40)],
                            macc.at[pl.ds(s * 640 + t * 40, 40)])
            return carry

        lax.fori_loop(0, 640 // 40, zslice, 0)
        pltpu.sync_copy(c16_hbm, c16v)
        plsc.subcore_barrier()

        def issue_a(cidx, slot, avX, saX):
            for t in range(K // 16):
                sl = pl.ds(t * 16, 16)
                ixa[slot, sl] = dstrv[cidx, sl] + goff
            return pltpu.async_copy(a_hbm.at[ixa.at[slot]], avX, saX)

        def issue_b(cidx):
            for t in range(K // 16):
                sl = pl.ds(t * 16, 16)
                ixb[sl] = (srcrv[cidx, sl] & 0xFFFFF) + goff
            return pltpu.async_copy(b_hbm.at[ixb], bv, sb)

        def compute(avX, cidx):
            def grp(g, gcarry):
                base = g * 16
                attr16 = lax.shift_right_logical(
                    srcrv[cidx, pl.ds(base, 16)], 20)
                for i in range(16):
                    r = base + i
                    a_i = attr16[i]
                    for t in range(D // 16):
                        sl = pl.ds(t * 16, 16)
                        avX[r, sl] = jnp.maximum(
                            avX[r, sl] + bv[r, sl] + c16v[a_i, sl], 0.0)
                return gcarry

            lax.fori_loop(0, K // 16, grp, 0)

        def block(b, carry):
            pltpu.sync_copy(dstr_hbm.at[c, s, b], dstrv)
            pltpu.sync_copy(srcr_hbm.at[c, s, b], srcrv)

            def pair(p, pcarry):
                c0 = 2 * p
                c1 = c0 + 1
                ga0 = issue_a(c0, 0, av0, sa0)
                ga1 = issue_a(c1, 1, av1, sa1)
                gb0 = issue_b(c0)
                ga0.wait()
                gb0.wait()
                compute(av0, c0)
                sc0 = pltpu.async_copy(av0, macc.at[dstrv.at[c0]], ss0,
                                       add=True)
                gb1 = issue_b(c1)
                ga1.wait()
                gb1.wait()
                compute(av1, c1)
                sc1 = pltpu.async_copy(av1, macc.at[dstrv.at[c1]], ss1,
                                       add=True)
                sc0.wait()
                sc1.wait()
                return pcarry

            lax.fori_loop(0, PAIRS, pair, 0)
            ct = BLK - 1
            ga = issue_a(ct, 0, av0, sa0)
            gb = issue_b(ct)
            ga.wait()
            gb.wait()
            compute(av0, ct)
            pltpu.sync_copy(av0, macc.at[dstrv.at[ct]], add=True)
            return carry

        lax.fori_loop(0, NBLOCKS, block, 0)
        plsc.subcore_barrier()

        def wb(t, carry):
            off = s * 640 + t * 40
            pltpu.sync_copy(macc.at[pl.ds(off, 40)], bv.at[pl.ds(0, 40)])
            pltpu.sync_copy(bv.at[pl.ds(0, 40)],
                            out_hbm.at[pl.ds(c * NP + off, 40)])
            return carry

        lax.fori_loop(0, 640 // 40, wb, 0)

    return k


def _msg_kernel(*a):
    return _make_msg_kernel()(*a)


def _make_flash(Bq, Bk):
    nk = NP // Bk

    def body(q_ref, kv_ref, o_ref, acc_ref, l_ref):
        j = pl.program_id(1)

        @pl.when(j == 0)
        def _():
            acc_ref[...] = jnp.zeros_like(acc_ref)
            l_ref[...] = jnp.zeros_like(l_ref)

        q = q_ref[...]
        kv = kv_ref[...]
        s = lax.dot_general(q, kv, (((1,), (1,)), ((), ())),
                            preferred_element_type=jnp.float32)
        col = j * Bk + lax.broadcasted_iota(jnp.int32, (Bq, Bk), 1)
        e = jnp.where(col < N_REAL, jnp.exp(s), 0.0)
        acc_ref[...] += jnp.dot(e, kv, preferred_element_type=jnp.float32)
        l_ref[...] += jnp.sum(e, axis=1, keepdims=True)

        @pl.when(j == nk - 1)
        def _():
            o_ref[...] = q - acc_ref[...] / l_ref[...]

    return pl.pallas_call(
        body,
        grid=(NP // Bq, nk),
        in_specs=[
            pl.BlockSpec((Bq, D), lambda i, j: (i, 0)),
            pl.BlockSpec((Bk, D), lambda i, j: (j, 0)),
        ],
        out_specs=pl.BlockSpec((Bq, D), lambda i, j: (i, 0)),
        out_shape=jax.ShapeDtypeStruct((NP, D), jnp.float32),
        scratch_shapes=[
            pltpu.VMEM((Bq, D), jnp.float32),
            pltpu.VMEM((Bq, 128), jnp.float32),
        ],
    )


_flash = _make_flash(512, 512)


def _make_gru_pool(Bg):
    ng = NP // Bg

    def body(m_ref, u_ref, h_ref, wim_ref, wiu_ref, whh_ref, bih_ref,
             bhh_ref, wg_ref, bg_ref, o_ref, num_ref, den_ref):
        i = pl.program_id(0)

        @pl.when(i == 0)
        def _():
            num_ref[...] = jnp.zeros_like(num_ref)
            den_ref[...] = jnp.zeros_like(den_ref)

        m = m_ref[...]
        u = u_ref[...]
        h = h_ref[...]
        gi = (jnp.dot(m, wim_ref[...], preferred_element_type=jnp.float32)
              + jnp.dot(u, wiu_ref[...], preferred_element_type=jnp.float32)
              + bih_ref[...])
        gh = (jnp.dot(h, whh_ref[...], preferred_element_type=jnp.float32)
              + bhh_ref[...])
        r = jax.nn.sigmoid(gi[:, :D] + gh[:, :D])
        z = jax.nn.sigmoid(gi[:, D:2 * D] + gh[:, D:2 * D])
        n = jnp.tanh(gi[:, 2 * D:] + r * gh[:, 2 * D:])
        hn = (1.0 - z) * n + z * h

        gate = jax.nn.sigmoid(
            jnp.dot(hn, wg_ref[...], preferred_element_type=jnp.float32)
            + bg_ref[...])
        rowid = i * Bg + lax.broadcasted_iota(jnp.int32, (Bg, 1), 0)
        eg = jnp.where(rowid < N_REAL, jnp.exp(gate), 0.0)
        num_ref[...] += jnp.sum(eg * hn, axis=0, keepdims=True)
        den_ref[...] += jnp.sum(eg)

        @pl.when(i == ng - 1)
        def _():
            o_ref[...] = num_ref[...] / den_ref[...]

    full = lambda i: (0, 0)
    return pl.pallas_call(
        body,
        grid=(ng,),
        in_specs=[
            pl.BlockSpec((Bg, D), lambda i: (i, 0)),
            pl.BlockSpec((Bg, D), lambda i: (i, 0)),
            pl.BlockSpec((Bg, D), lambda i: (i, 0)),
            pl.BlockSpec((D, 3 * D), full),
            pl.BlockSpec((D, 3 * D), full),
            pl.BlockSpec((D, 3 * D), full),
            pl.BlockSpec((1, 3 * D), full),
            pl.BlockSpec((1, 3 * D), full),
            pl.BlockSpec((D, 1), full),
            pl.BlockSpec((1, 1), full),
        ],
        out_specs=pl.BlockSpec((1, D), full),
        out_shape=jax.ShapeDtypeStruct((1, D), jnp.float32),
        scratch_shapes=[
            pltpu.VMEM((1, D), jnp.float32),
            pltpu.VMEM((1, 128), jnp.float32),
        ],
    )


_gru_pool = _make_gru_pool(512)


def kernel(x1, x2, edge_index1, edge_index2, edge_attr1, edge_attr2,
           embed, edge_embed, W_msg, b_msg, w_ih, w_hh, b_ih, b_hh,
           W_gate, b_gate):
    i32 = jnp.int32
    pad = jnp.zeros((NP - N_REAL,), i32)
    idx = jnp.concatenate([
        x1[:, 0].astype(i32), pad, x2[:, 0].astype(i32), pad])

    hp = _embed_gather(embed, idx)

    w1t = W_msg[:, :D].T
    w2t = W_msg[:, D:2 * D].T
    w3t = W_msg[:, 2 * D:].T
    a_all, b_all, c16 = _proj(hp, w1t, w2t, edge_embed,
                              w3t, b_msg.reshape(1, D))

    def edge_layout(ei, ea):
        dst = ei[1].astype(i32)
        srcp = ei[0].astype(i32) | (ea[:, 0].astype(i32) << 20)
        shp = (NS, NBLOCKS, BLK, K)
        return (dst.reshape(shp), srcp.reshape(shp))

    d1, s1 = edge_layout(edge_index1, edge_attr1)
    d2, s2 = edge_layout(edge_index2, edge_attr2)
    dst_raw = jnp.stack([d1, d2])
    src_pk = jnp.stack([s1, s2])

    m_all = _msg_kernel(a_all, b_all, c16, dst_raw, src_pk)

    h1p = hp[:NP]
    h2p = hp[NP:]
    u1 = _flash(h1p, h2p)
    u2 = _flash(h2p, h1p)

    wim = w_ih[:, :D].T
    wiu = w_ih[:, D:].T
    whh = w_hh.T
    bih = b_ih.reshape(1, 3 * D)
    bhh = b_hh.reshape(1, 3 * D)
    wg = W_gate.T
    bg = b_gate.reshape(1, 1)

    hg1 = _gru_pool(m_all[:NP], u1, h1p, wim, wiu, whh, bih, bhh, wg, bg)
    hg2 = _gru_pool(m_all[NP:], u2, h2p, wim, wiu, whh, bih, bhh, wg, bg)
    return (hg1, hg2)
